# baseline (speedup 1.0000x reference)
import functools

import jax
import jax.numpy as jnp
from jax import lax
from jax.experimental import pallas as pl
from jax.experimental.pallas import tpu as pltpu
from jax.experimental.pallas import tpu_sc as plsc

N = 8 * 512 * 512
NW = 32
PER_W = N // NW
CH = 16384
NCH = PER_W // CH
B = 2048
BF = float(B)


ROWS = 32


def _sc_stats(pred3, targ3):
    mesh = plsc.VectorSubcoreMesh(core_axis_name="c", subcore_axis_name="s")

    @functools.partial(
        pl.kernel,
        out_type=(
            jax.ShapeDtypeStruct((NW * B // 128, 128), jnp.float32),
            jax.ShapeDtypeStruct((NW, 16), jnp.float32),
        ),
        mesh=mesh,
        compiler_params=pltpu.CompilerParams(needs_layout_passes=False,
                                             use_tc_tiling_on_sc=True),
        scratch_types=[
            pltpu.VMEM((2, ROWS, 512), jnp.float32),
            pltpu.VMEM((2, ROWS, 512), jnp.int32),
            pltpu.VMEM((16 * B,), jnp.float32),
            pltpu.VMEM((B // 128, 128), jnp.float32),
            pltpu.VMEM((16,), jnp.float32),
            pltpu.SemaphoreType.DMA,
            pltpu.SemaphoreType.DMA,
            pltpu.SemaphoreType.DMA,
            pltpu.SemaphoreType.DMA,
        ],
    )
    def k(pred_hbm, targ_hbm, hist_out, scal_out, p_v, t_v, hist_v, fold_v,
          scal_v, sp0, sp1, st0, st1):
        cid = lax.axis_index("c")
        sid = lax.axis_index("s")
        wid = sid * 2 + cid
        img = wid // 4
        row0 = (wid % 4) * 128

        zero16 = jnp.zeros((16,), jnp.float32)
        ones16 = jnp.ones((16,), jnp.float32)
        lane_base = lax.iota(jnp.int32, 16) * B

        @plsc.parallel_loop(0, B, unroll=8)
        def _(i):
            hist_v[pl.ds(i * 16, 16)] = zero16

        psem = (sp0, sp1)
        tsem = (st0, st1)

        def start(j, buf):
            r = row0 + j * ROWS
            cp = pltpu.async_copy(pred_hbm.at[img, pl.ds(r, ROWS), :],
                                  p_v.at[buf], psem[buf])
            ct = pltpu.async_copy(targ_hbm.at[img, pl.ds(r, ROWS), :],
                                  t_v.at[buf], tsem[buf])
            return cp, ct

        pending = start(0, 0)
        accs = (zero16,) * 8
        for j in range(NCH):
            buf = j % 2
            pending[0].wait()
            pending[1].wait()
            if j + 1 < NCH:
                pending = start(j + 1, 1 - buf)

            def body(i, acc8):
                row = i >> 5
                colbase = (i & 31) * 16
                out = []
                for u in range(8):
                    p = p_v[buf, row, pl.ds(colbase + u * 16, 16)]
                    t = t_v[buf, row, pl.ds(colbase + u * 16, 16)]
                    tf = t.astype(jnp.float32)
                    out.append(acc8[u] + p * tf)
                    kf = jnp.minimum(p * BF, BF - 1.0)
                    idx = lane_base + kf.astype(jnp.int32)
                    plsc.addupdate_scatter(hist_v, [idx], ones16,
                                           mask=(t == 0))
                return tuple(out)

            accs = plsc.parallel_loop(0, ROWS * 32, 8, unroll=2,
                                      carry=accs)(body)

        sumq = (((accs[0] + accs[1]) + (accs[2] + accs[3]))
                + ((accs[4] + accs[5]) + (accs[6] + accs[7])))

        @plsc.parallel_loop(0, B // 16, unroll=2)
        def _(jj):
            acc = hist_v[pl.ds(jj * 16, 16)]
            for l in range(1, 16):
                acc = acc + hist_v[pl.ds(l * B + jj * 16, 16)]
            fold_v[jj >> 3, pl.ds((jj & 7) * 16, 16)] = acc

        scal_v[pl.ds(0, 16)] = sumq
        pltpu.sync_copy(fold_v, hist_out.at[pl.ds(wid * (B // 128), B // 128), :])
        pltpu.sync_copy(scal_v, scal_out.at[wid])

    return k(pred3, targ3)


def _tc_finish_body(h_ref, s_ref, o_ref):
    h3 = h_ref[...].reshape(NW, B // 128, 128)
    c2 = jnp.sum(h3, axis=0)
    sumq = jnp.sum(s_ref[...])

    r = B // 128
    ri = lax.broadcasted_iota(jnp.int32, (r, 128), 0).astype(jnp.float32)
    ci = lax.broadcasted_iota(jnp.int32, (r, 128), 1).astype(jnp.float32)
    ehat = 1.0 + ((ri * 128.0 + ci) + 0.5) / BF

    u = (lax.broadcasted_iota(jnp.int32, (128, 128), 0)
         <= lax.broadcasted_iota(jnp.int32, (128, 128), 1)).astype(jnp.float32)
    rowcs = lax.dot(c2, u, precision=lax.Precision.HIGHEST)
    rtot = jnp.sum(c2, axis=1, keepdims=True)
    strict = (lax.broadcasted_iota(jnp.int32, (r, r), 1)
              < lax.broadcasted_iota(jnp.int32, (r, r), 0)).astype(jnp.float32)
    offs = lax.dot(strict, rtot, precision=lax.Precision.HIGHEST)
    pfx = rowcs + offs

    nf = float(N)
    n0 = jnp.sum(rtot)
    g = nf - n0
    e1 = g - sumq
    e0m = jnp.max(jnp.where(c2 > 0.0, ehat, 0.0))
    rem = nf - pfx
    denom = jnp.maximum(rem * (rem + c2), 1.0)
    contrib = jnp.sum(g * ehat * c2 / denom)
    loss = contrib + e1 / nf + jnp.where(g == 0.0, e0m, 0.0)
    o_ref[...] = jnp.reshape(loss, (1, 1))


def _tc_finish(hist2d, scal):
    return pl.pallas_call(
        _tc_finish_body,
        out_shape=jax.ShapeDtypeStruct((1, 1), jnp.float32),
    )(hist2d, scal)


def kernel(pred, target):
    hist, scal = _sc_stats(pred, target)
    out = _tc_finish(hist, scal)
    return out.reshape(())


# speedup vs baseline: 1.1329x; 1.1329x over previous
import functools

import jax
import jax.numpy as jnp
from jax import lax
from jax.experimental import pallas as pl
from jax.experimental.pallas import tpu as pltpu
from jax.experimental.pallas import tpu_sc as plsc

N = 8 * 512 * 512
NW = 32
PER_W = N // NW
CH = 16384
NCH = PER_W // CH
B = 2048
BF = float(B)


ROWS = 32


def _sc_stats(pred3, targ3):
    mesh = plsc.VectorSubcoreMesh(core_axis_name="c", subcore_axis_name="s")

    @functools.partial(
        pl.kernel,
        out_type=(
            jax.ShapeDtypeStruct((NW * B // 128, 128), jnp.float32),
            jax.ShapeDtypeStruct((NW, 16), jnp.float32),
        ),
        mesh=mesh,
        compiler_params=pltpu.CompilerParams(needs_layout_passes=False,
                                             use_tc_tiling_on_sc=True),
        scratch_types=[
            pltpu.VMEM((2, ROWS, 512), jnp.float32),
            pltpu.VMEM((2, ROWS, 512), jnp.int32),
            pltpu.VMEM((16 * B,), jnp.float32),
            pltpu.VMEM((B // 128, 128), jnp.float32),
            pltpu.VMEM((16,), jnp.float32),
            pltpu.SemaphoreType.DMA,
            pltpu.SemaphoreType.DMA,
            pltpu.SemaphoreType.DMA,
            pltpu.SemaphoreType.DMA,
        ],
    )
    def k(pred_hbm, targ_hbm, hist_out, scal_out, p_v, t_v, hist_v, fold_v,
          scal_v, sp0, sp1, st0, st1):
        cid = lax.axis_index("c")
        sid = lax.axis_index("s")
        wid = sid * 2 + cid
        img = wid // 4
        row0 = (wid % 4) * 128

        zero16 = jnp.zeros((16,), jnp.float32)
        ones16 = jnp.ones((16,), jnp.float32)
        lane_base = lax.iota(jnp.int32, 16) * B

        @plsc.parallel_loop(0, B, unroll=8)
        def _(i):
            hist_v[pl.ds(i * 16, 16)] = zero16

        psem = (sp0, sp1)
        tsem = (st0, st1)

        def start(j, buf):
            r = row0 + j * ROWS
            cp = pltpu.async_copy(pred_hbm.at[img, pl.ds(r, ROWS), :],
                                  p_v.at[buf], psem[buf])
            ct = pltpu.async_copy(targ_hbm.at[img, pl.ds(r, ROWS), :],
                                  t_v.at[buf], tsem[buf])
            return cp, ct

        pending = start(0, 0)
        accs = (zero16,) * 8
        for j in range(NCH):
            buf = j % 2
            pending[0].wait()
            pending[1].wait()
            if j + 1 < NCH:
                pending = start(j + 1, 1 - buf)

            def body(i, acc8):
                row = i >> 5
                colbase = (i & 31) * 16
                out = []
                for u in range(8):
                    p = p_v[buf, row, pl.ds(colbase + u * 16, 16)]
                    t = t_v[buf, row, pl.ds(colbase + u * 16, 16)]
                    tf = t.astype(jnp.float32)
                    out.append(acc8[u] + p * tf)
                    kf = jnp.minimum(p * BF, BF - 1.0)
                    idx = lane_base + kf.astype(jnp.int32)
                    plsc.addupdate_scatter(hist_v, [idx], ones16,
                                           mask=(t == 0))
                return tuple(out)

            accs = plsc.parallel_loop(0, ROWS * 32, 8, carry=accs)(body)

        sumq = (((accs[0] + accs[1]) + (accs[2] + accs[3]))
                + ((accs[4] + accs[5]) + (accs[6] + accs[7])))

        @plsc.parallel_loop(0, B // 16, unroll=2)
        def _(jj):
            acc = hist_v[pl.ds(jj * 16, 16)]
            for l in range(1, 16):
                acc = acc + hist_v[pl.ds(l * B + jj * 16, 16)]
            fold_v[jj >> 3, pl.ds((jj & 7) * 16, 16)] = acc

        scal_v[pl.ds(0, 16)] = sumq
        pltpu.sync_copy(fold_v, hist_out.at[pl.ds(wid * (B // 128), B // 128), :])
        pltpu.sync_copy(scal_v, scal_out.at[wid])

    return k(pred3, targ3)


def _tc_finish_body(h_ref, s_ref, o_ref):
    h3 = h_ref[...].reshape(NW, B // 128, 128)
    c2 = jnp.sum(h3, axis=0)
    sumq = jnp.sum(s_ref[...])

    r = B // 128
    ri = lax.broadcasted_iota(jnp.int32, (r, 128), 0).astype(jnp.float32)
    ci = lax.broadcasted_iota(jnp.int32, (r, 128), 1).astype(jnp.float32)
    ehat = 1.0 + ((ri * 128.0 + ci) + 0.5) / BF

    u = (lax.broadcasted_iota(jnp.int32, (128, 128), 0)
         <= lax.broadcasted_iota(jnp.int32, (128, 128), 1)).astype(jnp.float32)
    rowcs = lax.dot(c2, u, precision=lax.Precision.HIGHEST)
    rtot = jnp.sum(c2, axis=1, keepdims=True)
    strict = (lax.broadcasted_iota(jnp.int32, (r, r), 1)
              < lax.broadcasted_iota(jnp.int32, (r, r), 0)).astype(jnp.float32)
    offs = lax.dot(strict, rtot, precision=lax.Precision.HIGHEST)
    pfx = rowcs + offs

    nf = float(N)
    n0 = jnp.sum(rtot)
    g = nf - n0
    e1 = g - sumq
    e0m = jnp.max(jnp.where(c2 > 0.0, ehat, 0.0))
    rem = nf - pfx
    denom = jnp.maximum(rem * (rem + c2), 1.0)
    contrib = jnp.sum(g * ehat * c2 / denom)
    loss = contrib + e1 / nf + jnp.where(g == 0.0, e0m, 0.0)
    o_ref[...] = jnp.reshape(loss, (1, 1))


def _tc_finish(hist2d, scal):
    return pl.pallas_call(
        _tc_finish_body,
        out_shape=jax.ShapeDtypeStruct((1, 1), jnp.float32),
    )(hist2d, scal)


def kernel(pred, target):
    hist, scal = _sc_stats(pred, target)
    out = _tc_finish(hist, scal)
    return out.reshape(())
